# baseline (speedup 1.0000x reference)
import jax
import jax.numpy as jnp
from jax.experimental import pallas as pl
from jax.experimental.pallas import tpu as pltpu

_TAU_DECAY = 10.0
_N_SPLIT = 2


def _body(d_ref, s_ref, w_ref, b_ref, o_ref, acc_ref):
    j = pl.program_id(1)
    s = s_ref[0]
    d = d_ref[...]
    fr = jax.lax.dot_general(
        d, s, (((1,), (0,)), ((), ())), preferred_element_type=jnp.float32
    )
    part = jax.lax.dot_general(
        fr, w_ref[...], (((1,), (1,)), ((), ())),
        preferred_element_type=jnp.float32,
    )

    @pl.when(j == 0)
    def _():
        acc_ref[...] = part

    @pl.when(j > 0)
    def _():
        acc_ref[...] += part

    @pl.when(j == _N_SPLIT - 1)
    def _():
        o_ref[0] = acc_ref[...] + b_ref[...]


def kernel(spike_trains, W, b):
    B, T, N = spike_trains.shape
    O = W.shape[0]
    Nb = N // _N_SPLIT
    decay = jnp.exp(-jnp.arange(T, dtype=spike_trains.dtype) / _TAU_DECAY)
    decay = (decay / decay.sum()).reshape(1, T)
    b2 = b.reshape(1, O)
    return pl.pallas_call(
        _body,
        grid=(B, _N_SPLIT),
        in_specs=[
            pl.BlockSpec((1, T), lambda i, j: (0, 0)),
            pl.BlockSpec((1, T, Nb), lambda i, j: (i, 0, j)),
            pl.BlockSpec((O, Nb), lambda i, j: (0, j)),
            pl.BlockSpec((1, O), lambda i, j: (0, 0)),
        ],
        out_specs=pl.BlockSpec((1, 1, O), lambda i, j: (i, 0, 0)),
        out_shape=jax.ShapeDtypeStruct((B, 1, O), spike_trains.dtype),
        scratch_shapes=[pltpu.VMEM((1, O), jnp.float32)],
        compiler_params=pltpu.CompilerParams(
            dimension_semantics=("parallel", "arbitrary"),
        ),
        name="spike_rate_readout",
    )(decay, spike_trains, W, b2).reshape(B, O)
